# baseline (speedup 1.0000x reference)
import jax
import jax.numpy as jnp
from jax.experimental import pallas as pl

B, K, Z, H = 4, 256, 128, 128
TI = 128


def _fused_kernel(zf_ref, zh_ref, p_ref, w1_ref, b1_ref, w2_ref, b2_ref,
                  wu_ref, bu_ref, out_ref):
    zf = zf_ref[0]
    zh = zh_ref[0]
    m2 = jnp.dot(zf, w2_ref[...], preferred_element_type=jnp.float32) + b2_ref[...]
    neg = jnp.float32(-jnp.inf)
    nm = jnp.where(p_ref[0] != 0, jnp.float32(0), neg)

    rows = []
    for i in range(TI):
        s = m2 + nm[:, i:i + 1]
        rows.append(jnp.max(s, axis=0, keepdims=True))
    M = jnp.concatenate(rows, axis=0)

    m1 = jnp.dot(zh, w1_ref[...], preferred_element_type=jnp.float32) + b1_ref[...]
    m = jnp.where(M == neg, neg, jax.nn.relu(m1 + M))
    acc = jnp.dot(zh, wu_ref[:Z], preferred_element_type=jnp.float32)
    acc = acc + jnp.dot(m, wu_ref[Z:], preferred_element_type=jnp.float32)
    out_ref[0] = jax.nn.relu(acc + bu_ref[...])


@jax.jit
def kernel(z, P, W_M1, b_M1, W_M2, b_M2, W_U, b_U):
    return pl.pallas_call(
        _fused_kernel,
        grid=(B, K // TI),
        in_specs=[
            pl.BlockSpec((1, K, Z), lambda b, h: (b, 0, 0)),
            pl.BlockSpec((1, TI, Z), lambda b, h: (b, h, 0)),
            pl.BlockSpec((1, K, TI), lambda b, h: (b, 0, h)),
            pl.BlockSpec((Z, Z), lambda b, h: (0, 0)),
            pl.BlockSpec((1, Z), lambda b, h: (0, 0)),
            pl.BlockSpec((Z, Z), lambda b, h: (0, 0)),
            pl.BlockSpec((1, Z), lambda b, h: (0, 0)),
            pl.BlockSpec((2 * Z, H), lambda b, h: (0, 0)),
            pl.BlockSpec((1, H), lambda b, h: (0, 0)),
        ],
        out_specs=pl.BlockSpec((1, TI, H), lambda b, h: (b, h, 0)),
        out_shape=jax.ShapeDtypeStruct((B, K, H), jnp.float32),
    )(z, z, P, W_M1, b_M1.reshape(1, Z), W_M2, b_M2.reshape(1, Z),
      W_U, b_U.reshape(1, H))


# speedup vs baseline: 1.0507x; 1.0507x over previous
import jax
import jax.numpy as jnp
from jax.experimental import pallas as pl

B, K, Z, H = 4, 256, 128, 128


def _fused_kernel(z_ref, p_ref, w1_ref, b1_ref, w2_ref, b2_ref,
                  wu_ref, bu_ref, out_ref):
    z = z_ref[0]
    m2 = jnp.dot(z, w2_ref[...], preferred_element_type=jnp.float32) + b2_ref[...]
    neg = jnp.float32(-jnp.inf)
    nm = jnp.where(p_ref[0] != 0, jnp.float32(0), neg)

    rows = []
    for i in range(K):
        s = m2 + nm[:, i:i + 1]
        rows.append(jnp.max(s, axis=0, keepdims=True))
    M = jnp.concatenate(rows, axis=0)

    m1 = jnp.dot(z, w1_ref[...], preferred_element_type=jnp.float32) + b1_ref[...]
    m = jnp.where(M == neg, neg, jax.nn.relu(m1 + M))
    acc = jnp.dot(z, wu_ref[:Z], preferred_element_type=jnp.float32)
    acc = acc + jnp.dot(m, wu_ref[Z:], preferred_element_type=jnp.float32)
    out_ref[0] = jax.nn.relu(acc + bu_ref[...])


@jax.jit
def kernel(z, P, W_M1, b_M1, W_M2, b_M2, W_U, b_U):
    return pl.pallas_call(
        _fused_kernel,
        grid=(B,),
        in_specs=[
            pl.BlockSpec((1, K, Z), lambda b: (b, 0, 0)),
            pl.BlockSpec((1, K, K), lambda b: (b, 0, 0)),
            pl.BlockSpec((Z, Z), lambda b: (0, 0)),
            pl.BlockSpec((1, Z), lambda b: (0, 0)),
            pl.BlockSpec((Z, Z), lambda b: (0, 0)),
            pl.BlockSpec((1, Z), lambda b: (0, 0)),
            pl.BlockSpec((2 * Z, H), lambda b: (0, 0)),
            pl.BlockSpec((1, H), lambda b: (0, 0)),
        ],
        out_specs=pl.BlockSpec((1, K, H), lambda b: (b, 0, 0)),
        out_shape=jax.ShapeDtypeStruct((B, K, H), jnp.float32),
    )(z, P, W_M1, b_M1.reshape(1, Z), W_M2, b_M2.reshape(1, Z),
      W_U, b_U.reshape(1, H))


# speedup vs baseline: 1.0560x; 1.0050x over previous
import jax
import jax.numpy as jnp
from jax.experimental import pallas as pl

B, K, Z, H = 4, 256, 128, 128


def _fused_kernel(z_ref, p_ref, w1_ref, b1_ref, w2_ref, b2_ref,
                  wu_ref, bu_ref, out_ref):
  for bb in range(2):
    z = z_ref[bb]
    m2 = jnp.dot(z, w2_ref[...], preferred_element_type=jnp.float32) + b2_ref[...]
    neg = jnp.float32(-jnp.inf)
    nm = jnp.where(p_ref[bb] != 0, jnp.float32(0), neg)

    rows = []
    for i in range(K):
        s = m2 + nm[:, i:i + 1]
        rows.append(jnp.max(s, axis=0, keepdims=True))
    M = jnp.concatenate(rows, axis=0)

    m1 = jnp.dot(z, w1_ref[...], preferred_element_type=jnp.float32) + b1_ref[...]
    m = jnp.where(M == neg, neg, jax.nn.relu(m1 + M))
    acc = jnp.dot(z, wu_ref[:Z], preferred_element_type=jnp.float32)
    acc = acc + jnp.dot(m, wu_ref[Z:], preferred_element_type=jnp.float32)
    out_ref[bb] = jax.nn.relu(acc + bu_ref[...])


@jax.jit
def kernel(z, P, W_M1, b_M1, W_M2, b_M2, W_U, b_U):
    return pl.pallas_call(
        _fused_kernel,
        grid=(B // 2,),
        in_specs=[
            pl.BlockSpec((2, K, Z), lambda b: (b, 0, 0)),
            pl.BlockSpec((2, K, K), lambda b: (b, 0, 0)),
            pl.BlockSpec((Z, Z), lambda b: (0, 0)),
            pl.BlockSpec((1, Z), lambda b: (0, 0)),
            pl.BlockSpec((Z, Z), lambda b: (0, 0)),
            pl.BlockSpec((1, Z), lambda b: (0, 0)),
            pl.BlockSpec((2 * Z, H), lambda b: (0, 0)),
            pl.BlockSpec((1, H), lambda b: (0, 0)),
        ],
        out_specs=pl.BlockSpec((2, K, H), lambda b: (b, 0, 0)),
        out_shape=jax.ShapeDtypeStruct((B, K, H), jnp.float32),
    )(z, P, W_M1, b_M1.reshape(1, Z), W_M2, b_M2.reshape(1, Z),
      W_U, b_U.reshape(1, H))
